# Initial kernel scaffold; baseline (speedup 1.0000x reference)
import jax
import jax.numpy as jnp
from jax.experimental import pallas as pl


def kernel(x, edge_index, batch, W1, b1, W2, b2, Wh1, bh1, Wh2, bh2):
    raise NotImplementedError("write your pallas kernel here")


# speedup vs baseline: 18.7174x; 18.7174x over previous
import functools

import jax
import jax.numpy as jnp
from jax import lax
from jax.experimental import pallas as pl
from jax.experimental.pallas import tpu as pltpu
from jax.experimental.pallas import tpu_sc as plsc

N_NODES = 10000
N_PAD = 10240
IN_CH = 128
HID = 64
NUM_GRAPHS = 128
N_EDGES = 320000
NW = 32
EPW = 10240
E_PAD = NW * EPW
CH = 128
NCH = EPW // CH
ZROWS = N_PAD // 16
OROWS = N_NODES // 16
BLK = 1000
GRID = N_NODES // BLK

_mesh = plsc.VectorSubcoreMesh(core_axis_name="c", subcore_axis_name="s")


def _deg_body(dst_hbm, z_hbm, out_hbm, dstc, ones_v, deg_sh):
    c = lax.axis_index("c")
    s = lax.axis_index("s")
    wid = s * 2 + c
    base = wid * EPW

    def _ones(i, carry):
        ones_v[pl.ds(pl.multiple_of(i * 16, 16), 16)] = jnp.ones((16,), jnp.float32)
        return carry
    lax.fori_loop(0, CH // 16, _ones, 0)

    pltpu.sync_copy(z_hbm, deg_sh.at[pl.ds(s * ZROWS, ZROWS)])
    plsc.subcore_barrier()

    def _step(j, carry):
        off = pl.multiple_of(base + j * CH, 8)
        pltpu.sync_copy(dst_hbm.at[pl.ds(off, CH)], dstc.at[0])
        pltpu.sync_copy(ones_v, deg_sh.at[dstc.at[0]], add=True)
        return carry
    lax.fori_loop(0, NCH, _step, 0)
    plsc.subcore_barrier()
    pltpu.sync_copy(deg_sh.at[pl.ds(s * ZROWS, ZROWS)],
                    out_hbm.at[pl.ds(c * N_PAD + s * ZROWS, ZROWS)])


_deg_call = pl.kernel(
    _deg_body,
    out_type=jax.ShapeDtypeStruct((2 * N_PAD,), jnp.float32),
    mesh=_mesh,
    scratch_types=[
        pltpu.VMEM((1, CH), jnp.int32),
        pltpu.VMEM((CH,), jnp.float32),
        pltpu.VMEM_SHARED((N_PAD,), jnp.float32),
    ],
)


def _msg_body(y_hbm, src_hbm, dst_hbm, z_hbm, out_hbm, srcc, dstc, rows,
              acc_sh, sem):
    c = lax.axis_index("c")
    s = lax.axis_index("s")
    wid = s * 2 + c
    base = wid * EPW

    pltpu.sync_copy(z_hbm, acc_sh.at[pl.ds(s * ZROWS, ZROWS)])
    plsc.subcore_barrier()

    def _step(j, carry):
        off = pl.multiple_of(base + j * CH, 8)
        pltpu.sync_copy(src_hbm.at[pl.ds(off, CH)], srcc)
        pltpu.sync_copy(dst_hbm.at[pl.ds(off, CH)], dstc.at[0])
        pltpu.async_copy(y_hbm.at[srcc], rows, sem).wait()
        pltpu.sync_copy(rows, acc_sh.at[dstc.at[0]], add=True)
        return carry
    lax.fori_loop(0, NCH, _step, 0)
    plsc.subcore_barrier()
    pltpu.sync_copy(acc_sh.at[pl.ds(s * ZROWS, ZROWS)],
                    out_hbm.at[c, pl.ds(s * ZROWS, ZROWS)])


_msg_call = pl.kernel(
    _msg_body,
    out_type=jax.ShapeDtypeStruct((2, N_PAD, HID), jnp.float32),
    mesh=_mesh,
    scratch_types=[
        pltpu.VMEM((CH,), jnp.int32),
        pltpu.VMEM((1, CH), jnp.int32),
        pltpu.VMEM((CH, HID), jnp.float32),
        pltpu.VMEM_SHARED((N_PAD, HID), jnp.float32),
        pltpu.SemaphoreType.DMA,
    ],
    compiler_params=pltpu.CompilerParams(use_tc_tiling_on_sc=False),
)


def _a_body(x_ref, w_ref, d0_ref, d1_ref, y_ref, dinv_ref):
    dinv = lax.rsqrt(d0_ref[...] + d1_ref[...] + 1.0)
    y_ref[...] = jnp.dot(x_ref[...], w_ref[...],
                         preferred_element_type=jnp.float32) * dinv
    dinv_ref[...] = dinv


def _dense_a(x, w1, dp0, dp1):
    return pl.pallas_call(
        _a_body,
        grid=(GRID,),
        in_specs=[
            pl.BlockSpec((BLK, IN_CH), lambda i: (i, 0)),
            pl.BlockSpec((IN_CH, HID), lambda i: (0, 0)),
            pl.BlockSpec((BLK, 1), lambda i: (i, 0)),
            pl.BlockSpec((BLK, 1), lambda i: (i, 0)),
        ],
        out_specs=[
            pl.BlockSpec((BLK, HID), lambda i: (i, 0)),
            pl.BlockSpec((BLK, 1), lambda i: (i, 0)),
        ],
        out_shape=[
            jax.ShapeDtypeStruct((N_NODES, HID), jnp.float32),
            jax.ShapeDtypeStruct((N_NODES, 1), jnp.float32),
        ],
    )(x, w1, dp0, dp1)


def _b_body(acc_ref, y1_ref, dinv_ref, b1_ref, w2_ref, y2_ref):
    dinv = dinv_ref[...]
    h = jnp.maximum(dinv * (acc_ref[0] + acc_ref[1] + y1_ref[...]) + b1_ref[...],
                    0.0)
    y2_ref[...] = jnp.dot(h, w2_ref[...],
                          preferred_element_type=jnp.float32) * dinv


def _dense_b(accp, y1, dinv, b1, w2):
    return pl.pallas_call(
        _b_body,
        grid=(GRID,),
        in_specs=[
            pl.BlockSpec((2, BLK, HID), lambda i: (0, i, 0)),
            pl.BlockSpec((BLK, HID), lambda i: (i, 0)),
            pl.BlockSpec((BLK, 1), lambda i: (i, 0)),
            pl.BlockSpec((1, HID), lambda i: (0, 0)),
            pl.BlockSpec((HID, HID), lambda i: (0, 0)),
        ],
        out_specs=pl.BlockSpec((BLK, HID), lambda i: (i, 0)),
        out_shape=jax.ShapeDtypeStruct((N_NODES, HID), jnp.float32),
    )(accp, y1, dinv, b1, w2)


def _c_body(acc_ref, y2_ref, dinv_ref, b2_ref, bat_ref, wh1_ref, bh1_ref,
            wh2_ref, bh2_ref, out_ref, sums, counts):
    i = pl.program_id(0)

    @pl.when(i == 0)
    def _():
        sums[...] = jnp.zeros_like(sums)
        counts[...] = jnp.zeros_like(counts)

    h = jnp.maximum(
        dinv_ref[...] * (acc_ref[0] + acc_ref[1] + y2_ref[...]) + b2_ref[...],
        0.0)
    onehot = (lax.broadcasted_iota(jnp.int32, (NUM_GRAPHS, BLK), 0)
              == bat_ref[0]).astype(jnp.float32)
    sums[...] += jnp.dot(onehot, h, preferred_element_type=jnp.float32)
    counts[...] += jnp.sum(onehot, axis=1, keepdims=True)

    @pl.when(i == pl.num_programs(0) - 1)
    def _():
        hg = sums[...] / jnp.maximum(counts[...], 1.0)
        z = jnp.maximum(
            jnp.dot(hg, wh1_ref[...], preferred_element_type=jnp.float32)
            + bh1_ref[...], 0.0)
        out_ref[...] = (jnp.dot(z, wh2_ref[...],
                                preferred_element_type=jnp.float32)
                        + bh2_ref[...])


def _dense_c(accp, y2, dinv, b2, bat, wh1, bh1, wh2, bh2):
    return pl.pallas_call(
        _c_body,
        grid=(GRID,),
        in_specs=[
            pl.BlockSpec((2, BLK, HID), lambda i: (0, i, 0)),
            pl.BlockSpec((BLK, HID), lambda i: (i, 0)),
            pl.BlockSpec((BLK, 1), lambda i: (i, 0)),
            pl.BlockSpec((1, HID), lambda i: (0, 0)),
            pl.BlockSpec((1, 1, BLK), lambda i: (i, 0, 0)),
            pl.BlockSpec((HID, HID // 2), lambda i: (0, 0)),
            pl.BlockSpec((1, HID // 2), lambda i: (0, 0)),
            pl.BlockSpec((HID // 2, 1), lambda i: (0, 0)),
            pl.BlockSpec((1, 1), lambda i: (0, 0)),
        ],
        out_specs=pl.BlockSpec((NUM_GRAPHS, 1), lambda i: (0, 0)),
        out_shape=jax.ShapeDtypeStruct((NUM_GRAPHS, 1), jnp.float32),
        scratch_shapes=[
            pltpu.VMEM((NUM_GRAPHS, HID), jnp.float32),
            pltpu.VMEM((NUM_GRAPHS, 1), jnp.float32),
        ],
    )(accp, y2, dinv, b2, bat, wh1, bh1, wh2, bh2)


def kernel(x, edge_index, batch, W1, b1, W2, b2, Wh1, bh1, Wh2, bh2):
    src = edge_index[0].astype(jnp.int32)
    dst = edge_index[1].astype(jnp.int32)
    npad = E_PAD - N_EDGES
    pad_ar = jnp.arange(npad, dtype=jnp.int32)
    srcp = jnp.concatenate([src, (pad_ar * 37) % N_NODES])
    dstp = jnp.concatenate([dst, N_NODES + pad_ar % (N_PAD - N_NODES)])
    z1 = jnp.zeros((ZROWS,), jnp.float32)
    z2 = jnp.zeros((ZROWS, HID), jnp.float32)

    degp = _deg_call(dstp, z1).reshape(2, N_PAD)
    dp0 = degp[0, :N_NODES, None]
    dp1 = degp[1, :N_NODES, None]
    y1, dinv = _dense_a(x, W1, dp0, dp1)
    accp1 = _msg_call(y1, srcp, dstp, z2)
    y2 = _dense_b(accp1, y1, dinv, b1.reshape(1, HID), W2)
    accp2 = _msg_call(y2, srcp, dstp, z2)
    out = _dense_c(accp2, y2, dinv, b2.reshape(1, HID),
                   batch.astype(jnp.int32).reshape(GRID, 1, BLK),
                   Wh1, bh1.reshape(1, HID // 2), Wh2, bh2.reshape(1, 1))
    return out[:, 0]


# speedup vs baseline: 41.9219x; 2.2397x over previous
import functools

import jax
import jax.numpy as jnp
from jax import lax
from jax.experimental import pallas as pl
from jax.experimental.pallas import tpu as pltpu
from jax.experimental.pallas import tpu_sc as plsc

N_NODES = 10000
N_PAD = 10240
IN_CH = 128
HID = 64
NUM_GRAPHS = 128
N_EDGES = 320000
NW = 32
EPW = 10240
E_PAD = NW * EPW
CH = 128
NCH = EPW // CH
ZROWS = N_PAD // 16
OROWS = N_NODES // 16
BLK = 1000
GRID = N_NODES // BLK

_mesh = plsc.VectorSubcoreMesh(core_axis_name="c", subcore_axis_name="s")


def _deg_body(dst_hbm, z_hbm, out_hbm, dstb, ones_v, deg_sh, sem):
    c = lax.axis_index("c")
    s = lax.axis_index("s")
    wid = s * 2 + c

    def _ones(i, carry):
        ones_v[pl.ds(pl.multiple_of(i * 16, 16), 16)] = jnp.ones((16,), jnp.float32)
        return carry
    lax.fori_loop(0, CH // 16, _ones, 0)

    pltpu.sync_copy(dst_hbm.at[wid], dstb)
    pltpu.sync_copy(z_hbm, deg_sh.at[pl.ds(s * ZROWS, ZROWS)])
    plsc.subcore_barrier()

    def _fire(j, carry):
        pltpu.async_copy(ones_v, deg_sh.at[dstb.at[j]], sem, add=True)
        return carry
    lax.fori_loop(0, NCH, _fire, 0)

    def _drain(j, carry):
        pltpu.make_async_copy(ones_v, deg_sh.at[dstb.at[j]], sem).wait()
        return carry
    lax.fori_loop(0, NCH, _drain, 0)
    plsc.subcore_barrier()
    pltpu.sync_copy(deg_sh.at[pl.ds(s * ZROWS, ZROWS)],
                    out_hbm.at[pl.ds(c * N_PAD + s * ZROWS, ZROWS)])


_deg_call = pl.kernel(
    _deg_body,
    out_type=jax.ShapeDtypeStruct((2 * N_PAD,), jnp.float32),
    mesh=_mesh,
    scratch_types=[
        pltpu.VMEM((NCH, CH), jnp.int32),
        pltpu.VMEM((CH,), jnp.float32),
        pltpu.VMEM_SHARED((N_PAD,), jnp.float32),
        pltpu.SemaphoreType.DMA,
    ],
)


NBUF = 4
NWAVES = NCH // NBUF


def _msg_body(y_hbm, src_hbm, dst_hbm, z_hbm, out_hbm, srcb, dstb, rows,
              acc_sh, gsems, ssems):
    c = lax.axis_index("c")
    s = lax.axis_index("s")
    wid = s * 2 + c

    pltpu.sync_copy(src_hbm.at[wid], srcb)
    pltpu.sync_copy(dst_hbm.at[wid], dstb)
    pltpu.sync_copy(z_hbm, acc_sh.at[pl.ds(s * ZROWS, ZROWS)])
    plsc.subcore_barrier()

    def gstart(j, b):
        pltpu.async_copy(y_hbm.at[srcb.at[j]], rows.at[b], gsems[b])

    def gwait(j, b):
        pltpu.make_async_copy(y_hbm.at[srcb.at[j]], rows.at[b],
                              gsems[b]).wait()

    def sstart(j, b):
        pltpu.async_copy(rows.at[b], acc_sh.at[dstb.at[j]], ssems[b],
                         add=True)

    def swait(j, b):
        pltpu.make_async_copy(rows.at[b], acc_sh.at[dstb.at[j]],
                              ssems[b]).wait()

    for b in range(NBUF):
        gstart(b, b)

    def _wave(g, carry):
        j0 = g * NBUF
        for b in range(NBUF):
            gwait(j0 + b, b)
            sstart(j0 + b, b)
        for b in range(NBUF):
            swait(j0 + b, b)

            @pl.when(g + 1 < NWAVES)
            def _():
                gstart(j0 + NBUF + b, b)
        return carry
    lax.fori_loop(0, NWAVES, _wave, 0)
    plsc.subcore_barrier()
    pltpu.sync_copy(acc_sh.at[pl.ds(s * ZROWS, ZROWS)],
                    out_hbm.at[c, pl.ds(s * ZROWS, ZROWS)])


_msg_call = pl.kernel(
    _msg_body,
    out_type=jax.ShapeDtypeStruct((2, N_PAD, HID), jnp.float32),
    mesh=_mesh,
    scratch_types=[
        pltpu.VMEM((NCH, CH), jnp.int32),
        pltpu.VMEM((NCH, CH), jnp.int32),
        pltpu.VMEM((NBUF, CH, HID), jnp.float32),
        pltpu.VMEM_SHARED((N_PAD, HID), jnp.float32),
        [pltpu.SemaphoreType.DMA] * NBUF,
        [pltpu.SemaphoreType.DMA] * NBUF,
    ],
    compiler_params=pltpu.CompilerParams(use_tc_tiling_on_sc=False),
)


def _a_body(x_ref, w_ref, d0_ref, d1_ref, y_ref, dinv_ref):
    dinv = lax.rsqrt(d0_ref[...] + d1_ref[...] + 1.0)
    y_ref[...] = jnp.dot(x_ref[...], w_ref[...],
                         preferred_element_type=jnp.float32) * dinv
    dinv_ref[...] = dinv


def _dense_a(x, w1, dp0, dp1):
    return pl.pallas_call(
        _a_body,
        grid=(GRID,),
        in_specs=[
            pl.BlockSpec((BLK, IN_CH), lambda i: (i, 0)),
            pl.BlockSpec((IN_CH, HID), lambda i: (0, 0)),
            pl.BlockSpec((BLK, 1), lambda i: (i, 0)),
            pl.BlockSpec((BLK, 1), lambda i: (i, 0)),
        ],
        out_specs=[
            pl.BlockSpec((BLK, HID), lambda i: (i, 0)),
            pl.BlockSpec((BLK, 1), lambda i: (i, 0)),
        ],
        out_shape=[
            jax.ShapeDtypeStruct((N_NODES, HID), jnp.float32),
            jax.ShapeDtypeStruct((N_NODES, 1), jnp.float32),
        ],
    )(x, w1, dp0, dp1)


def _b_body(acc_ref, y1_ref, dinv_ref, b1_ref, w2_ref, y2_ref):
    dinv = dinv_ref[...]
    h = jnp.maximum(dinv * (acc_ref[0] + acc_ref[1] + y1_ref[...]) + b1_ref[...],
                    0.0)
    y2_ref[...] = jnp.dot(h, w2_ref[...],
                          preferred_element_type=jnp.float32) * dinv


def _dense_b(accp, y1, dinv, b1, w2):
    return pl.pallas_call(
        _b_body,
        grid=(GRID,),
        in_specs=[
            pl.BlockSpec((2, BLK, HID), lambda i: (0, i, 0)),
            pl.BlockSpec((BLK, HID), lambda i: (i, 0)),
            pl.BlockSpec((BLK, 1), lambda i: (i, 0)),
            pl.BlockSpec((1, HID), lambda i: (0, 0)),
            pl.BlockSpec((HID, HID), lambda i: (0, 0)),
        ],
        out_specs=pl.BlockSpec((BLK, HID), lambda i: (i, 0)),
        out_shape=jax.ShapeDtypeStruct((N_NODES, HID), jnp.float32),
    )(accp, y1, dinv, b1, w2)


def _c_body(acc_ref, y2_ref, dinv_ref, b2_ref, bat_ref, wh1_ref, bh1_ref,
            wh2_ref, bh2_ref, out_ref, sums, counts):
    i = pl.program_id(0)

    @pl.when(i == 0)
    def _():
        sums[...] = jnp.zeros_like(sums)
        counts[...] = jnp.zeros_like(counts)

    h = jnp.maximum(
        dinv_ref[...] * (acc_ref[0] + acc_ref[1] + y2_ref[...]) + b2_ref[...],
        0.0)
    onehot = (lax.broadcasted_iota(jnp.int32, (NUM_GRAPHS, BLK), 0)
              == bat_ref[0]).astype(jnp.float32)
    sums[...] += jnp.dot(onehot, h, preferred_element_type=jnp.float32)
    counts[...] += jnp.sum(onehot, axis=1, keepdims=True)

    @pl.when(i == pl.num_programs(0) - 1)
    def _():
        hg = sums[...] / jnp.maximum(counts[...], 1.0)
        z = jnp.maximum(
            jnp.dot(hg, wh1_ref[...], preferred_element_type=jnp.float32)
            + bh1_ref[...], 0.0)
        out_ref[...] = (jnp.dot(z, wh2_ref[...],
                                preferred_element_type=jnp.float32)
                        + bh2_ref[...])


def _dense_c(accp, y2, dinv, b2, bat, wh1, bh1, wh2, bh2):
    return pl.pallas_call(
        _c_body,
        grid=(GRID,),
        in_specs=[
            pl.BlockSpec((2, BLK, HID), lambda i: (0, i, 0)),
            pl.BlockSpec((BLK, HID), lambda i: (i, 0)),
            pl.BlockSpec((BLK, 1), lambda i: (i, 0)),
            pl.BlockSpec((1, HID), lambda i: (0, 0)),
            pl.BlockSpec((1, 1, BLK), lambda i: (i, 0, 0)),
            pl.BlockSpec((HID, HID // 2), lambda i: (0, 0)),
            pl.BlockSpec((1, HID // 2), lambda i: (0, 0)),
            pl.BlockSpec((HID // 2, 1), lambda i: (0, 0)),
            pl.BlockSpec((1, 1), lambda i: (0, 0)),
        ],
        out_specs=pl.BlockSpec((NUM_GRAPHS, 1), lambda i: (0, 0)),
        out_shape=jax.ShapeDtypeStruct((NUM_GRAPHS, 1), jnp.float32),
        scratch_shapes=[
            pltpu.VMEM((NUM_GRAPHS, HID), jnp.float32),
            pltpu.VMEM((NUM_GRAPHS, 1), jnp.float32),
        ],
    )(accp, y2, dinv, b2, bat, wh1, bh1, wh2, bh2)


def kernel(x, edge_index, batch, W1, b1, W2, b2, Wh1, bh1, Wh2, bh2):
    src = edge_index[0].astype(jnp.int32)
    dst = edge_index[1].astype(jnp.int32)
    npad = E_PAD - N_EDGES
    pad_ar = jnp.arange(npad, dtype=jnp.int32)
    srcp = jnp.concatenate([src, (pad_ar * 37) % N_NODES]).reshape(NW, NCH, CH)
    dstp = jnp.concatenate(
        [dst, N_NODES + pad_ar % (N_PAD - N_NODES)]).reshape(NW, NCH, CH)
    z1 = jnp.zeros((ZROWS,), jnp.float32)
    z2 = jnp.zeros((ZROWS, HID), jnp.float32)

    degp = _deg_call(dstp, z1).reshape(2, N_PAD)
    dp0 = degp[0, :N_NODES, None]
    dp1 = degp[1, :N_NODES, None]
    y1, dinv = _dense_a(x, W1, dp0, dp1)
    accp1 = _msg_call(y1, srcp, dstp, z2)
    y2 = _dense_b(accp1, y1, dinv, b1.reshape(1, HID), W2)
    accp2 = _msg_call(y2, srcp, dstp, z2)
    out = _dense_c(accp2, y2, dinv, b2.reshape(1, HID),
                   batch.astype(jnp.int32).reshape(GRID, 1, BLK),
                   Wh1, bh1.reshape(1, HID // 2), Wh2, bh2.reshape(1, 1))
    return out[:, 0]


# speedup vs baseline: 44.6207x; 1.0644x over previous
import functools

import numpy as np

import jax
import jax.numpy as jnp
from jax import lax
from jax.experimental import pallas as pl
from jax.experimental.pallas import tpu as pltpu
from jax.experimental.pallas import tpu_sc as plsc

N_NODES = 10000
N_PAD = 10240
IN_CH = 128
HID = 64
NUM_GRAPHS = 128
N_EDGES = 320000
NW = 32
EPW = 10240
E_PAD = NW * EPW
CH = 128
NCH = EPW // CH
ZROWS = N_PAD // 16
OROWS = N_NODES // 16
BLK = 2000
GRID = N_NODES // BLK

_mesh = plsc.VectorSubcoreMesh(core_axis_name="c", subcore_axis_name="s")

_NPAD_E = E_PAD - N_EDGES
_PAD_SRC = np.asarray((np.arange(_NPAD_E) * 37) % N_NODES, dtype=np.int32)
_PAD_DST = np.asarray(N_NODES + np.arange(_NPAD_E) % (N_PAD - N_NODES),
                      dtype=np.int32)


def _deg_body(dst_hbm, z_hbm, out_hbm, dstb, ones_v, deg_sh, sem):
    c = lax.axis_index("c")
    s = lax.axis_index("s")
    wid = s * 2 + c

    def _ones(i, carry):
        ones_v[pl.ds(pl.multiple_of(i * 16, 16), 16)] = jnp.ones((16,), jnp.float32)
        return carry
    lax.fori_loop(0, CH // 16, _ones, 0)

    pltpu.sync_copy(dst_hbm.at[wid], dstb)
    pltpu.sync_copy(z_hbm, deg_sh.at[pl.ds(s * ZROWS, ZROWS)])
    plsc.subcore_barrier()

    def _fire(j, carry):
        pltpu.async_copy(ones_v, deg_sh.at[dstb.at[j]], sem, add=True)
        return carry
    lax.fori_loop(0, NCH, _fire, 0)

    def _drain(j, carry):
        pltpu.make_async_copy(ones_v, deg_sh.at[dstb.at[j]], sem).wait()
        return carry
    lax.fori_loop(0, NCH, _drain, 0)
    plsc.subcore_barrier()
    pltpu.sync_copy(deg_sh.at[pl.ds(s * ZROWS, ZROWS)],
                    out_hbm.at[pl.ds(c * N_PAD + s * ZROWS, ZROWS)])


_deg_call = pl.kernel(
    _deg_body,
    out_type=jax.ShapeDtypeStruct((2 * N_PAD,), jnp.float32),
    mesh=_mesh,
    scratch_types=[
        pltpu.VMEM((NCH, CH), jnp.int32),
        pltpu.VMEM((CH,), jnp.float32),
        pltpu.VMEM_SHARED((N_PAD,), jnp.float32),
        pltpu.SemaphoreType.DMA,
    ],
)


NBUF = 8
NWAVES = NCH // NBUF


def _msg_body(y_hbm, src_hbm, dst_hbm, z_hbm, out_hbm, srcb, dstb, rows,
              acc_sh, gsems, ssems):
    c = lax.axis_index("c")
    s = lax.axis_index("s")
    wid = s * 2 + c

    pltpu.sync_copy(src_hbm.at[wid], srcb)
    pltpu.sync_copy(dst_hbm.at[wid], dstb)
    pltpu.sync_copy(z_hbm, acc_sh.at[pl.ds(s * ZROWS, ZROWS)])
    plsc.subcore_barrier()

    def gstart(j, b):
        pltpu.async_copy(y_hbm.at[srcb.at[j]], rows.at[b], gsems[b])

    def gwait(j, b):
        pltpu.make_async_copy(y_hbm.at[srcb.at[j]], rows.at[b],
                              gsems[b]).wait()

    def sstart(j, b):
        pltpu.async_copy(rows.at[b], acc_sh.at[dstb.at[j]], ssems[b],
                         add=True)

    def swait(j, b):
        pltpu.make_async_copy(rows.at[b], acc_sh.at[dstb.at[j]],
                              ssems[b]).wait()

    for b in range(NBUF):
        gstart(b, b)

    def _wave(g, carry):
        j0 = g * NBUF
        for b in range(NBUF):
            gwait(j0 + b, b)
            sstart(j0 + b, b)
        for b in range(NBUF):
            swait(j0 + b, b)

            @pl.when(g + 1 < NWAVES)
            def _():
                gstart(j0 + NBUF + b, b)
        return carry
    lax.fori_loop(0, NWAVES, _wave, 0)
    plsc.subcore_barrier()
    pltpu.sync_copy(acc_sh.at[pl.ds(s * ZROWS, ZROWS)],
                    out_hbm.at[c, pl.ds(s * ZROWS, ZROWS)])


_msg_call = pl.kernel(
    _msg_body,
    out_type=jax.ShapeDtypeStruct((2, N_PAD, HID), jnp.float32),
    mesh=_mesh,
    scratch_types=[
        pltpu.VMEM((NCH, CH), jnp.int32),
        pltpu.VMEM((NCH, CH), jnp.int32),
        pltpu.VMEM((NBUF, CH, HID), jnp.float32),
        pltpu.VMEM_SHARED((N_PAD, HID), jnp.float32),
        [pltpu.SemaphoreType.DMA] * NBUF,
        [pltpu.SemaphoreType.DMA] * NBUF,
    ],
    compiler_params=pltpu.CompilerParams(use_tc_tiling_on_sc=False),
)


def _a_body(x_ref, w_ref, d0_ref, d1_ref, y_ref, dinv_ref):
    dinv = lax.rsqrt(d0_ref[...] + d1_ref[...] + 1.0)
    y_ref[...] = jnp.dot(x_ref[...], w_ref[...],
                         preferred_element_type=jnp.float32) * dinv
    dinv_ref[...] = dinv


def _dense_a(x, w1, dp0, dp1):
    return pl.pallas_call(
        _a_body,
        grid=(GRID,),
        in_specs=[
            pl.BlockSpec((BLK, IN_CH), lambda i: (i, 0)),
            pl.BlockSpec((IN_CH, HID), lambda i: (0, 0)),
            pl.BlockSpec((BLK, 1), lambda i: (i, 0)),
            pl.BlockSpec((BLK, 1), lambda i: (i, 0)),
        ],
        out_specs=[
            pl.BlockSpec((BLK, HID), lambda i: (i, 0)),
            pl.BlockSpec((BLK, 1), lambda i: (i, 0)),
        ],
        out_shape=[
            jax.ShapeDtypeStruct((N_NODES, HID), jnp.float32),
            jax.ShapeDtypeStruct((N_NODES, 1), jnp.float32),
        ],
    )(x, w1, dp0, dp1)


def _b_body(acc_ref, y1_ref, dinv_ref, b1_ref, w2_ref, y2_ref):
    dinv = dinv_ref[...]
    h = jnp.maximum(dinv * (acc_ref[0] + acc_ref[1] + y1_ref[...]) + b1_ref[...],
                    0.0)
    y2_ref[...] = jnp.dot(h, w2_ref[...],
                          preferred_element_type=jnp.float32) * dinv


def _dense_b(accp, y1, dinv, b1, w2):
    return pl.pallas_call(
        _b_body,
        grid=(GRID,),
        in_specs=[
            pl.BlockSpec((2, BLK, HID), lambda i: (0, i, 0)),
            pl.BlockSpec((BLK, HID), lambda i: (i, 0)),
            pl.BlockSpec((BLK, 1), lambda i: (i, 0)),
            pl.BlockSpec((1, HID), lambda i: (0, 0)),
            pl.BlockSpec((HID, HID), lambda i: (0, 0)),
        ],
        out_specs=pl.BlockSpec((BLK, HID), lambda i: (i, 0)),
        out_shape=jax.ShapeDtypeStruct((N_NODES, HID), jnp.float32),
    )(accp, y1, dinv, b1, w2)


def _c_body(acc_ref, y2_ref, dinv_ref, b2_ref, bat_ref, wh1_ref, bh1_ref,
            wh2_ref, bh2_ref, out_ref, sums, counts):
    i = pl.program_id(0)

    @pl.when(i == 0)
    def _():
        sums[...] = jnp.zeros_like(sums)
        counts[...] = jnp.zeros_like(counts)

    h = jnp.maximum(
        dinv_ref[...] * (acc_ref[0] + acc_ref[1] + y2_ref[...]) + b2_ref[...],
        0.0)
    onehot = (lax.broadcasted_iota(jnp.int32, (NUM_GRAPHS, BLK), 0)
              == bat_ref[0]).astype(jnp.float32)
    sums[...] += jnp.dot(onehot, h, preferred_element_type=jnp.float32)
    counts[...] += jnp.sum(onehot, axis=1, keepdims=True)

    @pl.when(i == pl.num_programs(0) - 1)
    def _():
        hg = sums[...] / jnp.maximum(counts[...], 1.0)
        z = jnp.maximum(
            jnp.dot(hg, wh1_ref[...], preferred_element_type=jnp.float32)
            + bh1_ref[...], 0.0)
        out_ref[...] = (jnp.dot(z, wh2_ref[...],
                                preferred_element_type=jnp.float32)
                        + bh2_ref[...])


def _dense_c(accp, y2, dinv, b2, bat, wh1, bh1, wh2, bh2):
    return pl.pallas_call(
        _c_body,
        grid=(GRID,),
        in_specs=[
            pl.BlockSpec((2, BLK, HID), lambda i: (0, i, 0)),
            pl.BlockSpec((BLK, HID), lambda i: (i, 0)),
            pl.BlockSpec((BLK, 1), lambda i: (i, 0)),
            pl.BlockSpec((1, HID), lambda i: (0, 0)),
            pl.BlockSpec((1, 1, BLK), lambda i: (i, 0, 0)),
            pl.BlockSpec((HID, HID // 2), lambda i: (0, 0)),
            pl.BlockSpec((1, HID // 2), lambda i: (0, 0)),
            pl.BlockSpec((HID // 2, 1), lambda i: (0, 0)),
            pl.BlockSpec((1, 1), lambda i: (0, 0)),
        ],
        out_specs=pl.BlockSpec((NUM_GRAPHS, 1), lambda i: (0, 0)),
        out_shape=jax.ShapeDtypeStruct((NUM_GRAPHS, 1), jnp.float32),
        scratch_shapes=[
            pltpu.VMEM((NUM_GRAPHS, HID), jnp.float32),
            pltpu.VMEM((NUM_GRAPHS, 1), jnp.float32),
        ],
    )(accp, y2, dinv, b2, bat, wh1, bh1, wh2, bh2)


def kernel(x, edge_index, batch, W1, b1, W2, b2, Wh1, bh1, Wh2, bh2):
    src = edge_index[0].astype(jnp.int32)
    dst = edge_index[1].astype(jnp.int32)
    srcp = jnp.concatenate([src, _PAD_SRC]).reshape(NW, NCH, CH)
    dstp = jnp.concatenate([dst, _PAD_DST]).reshape(NW, NCH, CH)
    z1 = jnp.zeros((ZROWS,), jnp.float32)
    z2 = jnp.zeros((ZROWS, HID), jnp.float32)

    degp = _deg_call(dstp, z1).reshape(2, N_PAD)
    dp0 = degp[0, :N_NODES, None]
    dp1 = degp[1, :N_NODES, None]
    y1, dinv = _dense_a(x, W1, dp0, dp1)
    accp1 = _msg_call(y1, srcp, dstp, z2)
    y2 = _dense_b(accp1, y1, dinv, b1.reshape(1, HID), W2)
    accp2 = _msg_call(y2, srcp, dstp, z2)
    out = _dense_c(accp2, y2, dinv, b2.reshape(1, HID),
                   batch.astype(jnp.int32).reshape(GRID, 1, BLK),
                   Wh1, bh1.reshape(1, HID // 2), Wh2, bh2.reshape(1, 1))
    return out[:, 0]


# speedup vs baseline: 45.2952x; 1.0151x over previous
import functools

import numpy as np

import jax
import jax.numpy as jnp
from jax import lax
from jax.experimental import pallas as pl
from jax.experimental.pallas import tpu as pltpu
from jax.experimental.pallas import tpu_sc as plsc

N_NODES = 10000
N_PAD = 10240
IN_CH = 128
HID = 64
NUM_GRAPHS = 128
N_EDGES = 320000
NW = 32
EPW = 10240
E_PAD = NW * EPW
CH = 128
NCH = EPW // CH
ZROWS = N_PAD // 16
OROWS = N_NODES // 16
BLK = 2000
GRID = N_NODES // BLK

_mesh = plsc.VectorSubcoreMesh(core_axis_name="c", subcore_axis_name="s")

_NPAD_E = E_PAD - N_EDGES
_PAD_SRC = np.asarray((np.arange(_NPAD_E) * 37) % N_NODES, dtype=np.int32)
_PAD_DST = np.asarray(N_NODES + np.arange(_NPAD_E) % (N_PAD - N_NODES),
                      dtype=np.int32)


def _deg_body(dst_hbm, z_hbm, out_hbm, dstb, ones_v, deg_sh, sem):
    c = lax.axis_index("c")
    s = lax.axis_index("s")
    wid = s * 2 + c

    def _ones(i, carry):
        ones_v[pl.ds(pl.multiple_of(i * 16, 16), 16)] = jnp.ones((16,), jnp.float32)
        return carry
    lax.fori_loop(0, CH // 16, _ones, 0)

    pltpu.sync_copy(dst_hbm.at[wid], dstb)
    pltpu.sync_copy(z_hbm, deg_sh.at[pl.ds(s * ZROWS, ZROWS)])
    plsc.subcore_barrier()

    def _fire(j, carry):
        pltpu.async_copy(ones_v, deg_sh.at[dstb.at[j]], sem, add=True)
        return carry
    lax.fori_loop(0, NCH, _fire, 0)

    def _drain(j, carry):
        pltpu.make_async_copy(ones_v, deg_sh.at[dstb.at[j]], sem).wait()
        return carry
    lax.fori_loop(0, NCH, _drain, 0)
    plsc.subcore_barrier()
    pltpu.sync_copy(deg_sh.at[pl.ds(s * ZROWS, ZROWS)],
                    out_hbm.at[pl.ds(c * N_PAD + s * ZROWS, ZROWS)])


_deg_call = pl.kernel(
    _deg_body,
    out_type=jax.ShapeDtypeStruct((2 * N_PAD,), jnp.float32),
    mesh=_mesh,
    scratch_types=[
        pltpu.VMEM((NCH, CH), jnp.int32),
        pltpu.VMEM((CH,), jnp.float32),
        pltpu.VMEM_SHARED((N_PAD,), jnp.float32),
        pltpu.SemaphoreType.DMA,
    ],
)


NBUF = 8
NWAVES = NCH // NBUF


def _msg_body(y_hbm, src_hbm, dst_hbm, z_hbm, out_hbm, srcb, dstb, rows,
              acc_sh, gsems, ssems):
    c = lax.axis_index("c")
    s = lax.axis_index("s")
    wid = s * 2 + c

    pltpu.sync_copy(src_hbm.at[wid], srcb)
    pltpu.sync_copy(dst_hbm.at[wid], dstb)
    pltpu.sync_copy(z_hbm, acc_sh.at[pl.ds(s * ZROWS, ZROWS)])
    plsc.subcore_barrier()

    def gstart(j, b):
        pltpu.async_copy(y_hbm.at[srcb.at[j]], rows.at[b], gsems[b])

    def gwait(j, b):
        pltpu.make_async_copy(y_hbm.at[srcb.at[j]], rows.at[b],
                              gsems[b]).wait()

    def sstart(j, b):
        pltpu.async_copy(rows.at[b], acc_sh.at[dstb.at[j]], ssems[b],
                         add=True)

    def swait(j, b):
        pltpu.make_async_copy(rows.at[b], acc_sh.at[dstb.at[j]],
                              ssems[b]).wait()

    for b in range(NBUF):
        gstart(b, b)

    def _wave(g, carry):
        j0 = g * NBUF
        for b in range(NBUF):
            gwait(j0 + b, b)
            sstart(j0 + b, b)
        for b in range(NBUF):
            swait(j0 + b, b)

            @pl.when(g + 1 < NWAVES)
            def _():
                gstart(j0 + NBUF + b, b)
        return carry
    lax.fori_loop(0, NWAVES, _wave, 0)
    plsc.subcore_barrier()
    pltpu.sync_copy(acc_sh.at[pl.ds(s * ZROWS, ZROWS)],
                    out_hbm.at[c, pl.ds(s * ZROWS, ZROWS)])


_msg_call = pl.kernel(
    _msg_body,
    out_type=jax.ShapeDtypeStruct((2, N_PAD, HID), jnp.float32),
    mesh=_mesh,
    scratch_types=[
        pltpu.VMEM((NCH, CH), jnp.int32),
        pltpu.VMEM((NCH, CH), jnp.int32),
        pltpu.VMEM((NBUF, CH, HID), jnp.float32),
        pltpu.VMEM_SHARED((N_PAD, HID), jnp.float32),
        [pltpu.SemaphoreType.DMA] * NBUF,
        [pltpu.SemaphoreType.DMA] * NBUF,
    ],
    compiler_params=pltpu.CompilerParams(use_tc_tiling_on_sc=False),
)


def _a_body(x_ref, w_ref, dinv_ref, y_ref):
    y_ref[...] = jnp.dot(x_ref[...], w_ref[...],
                         preferred_element_type=jnp.float32) * dinv_ref[...]


def _dense_a(x, w1, dinvb):
    return pl.pallas_call(
        _a_body,
        grid=(GRID,),
        in_specs=[
            pl.BlockSpec((BLK, IN_CH), lambda i: (i, 0)),
            pl.BlockSpec((IN_CH, HID), lambda i: (0, 0)),
            pl.BlockSpec((BLK, HID), lambda i: (i, 0)),
        ],
        out_specs=pl.BlockSpec((BLK, HID), lambda i: (i, 0)),
        out_shape=jax.ShapeDtypeStruct((N_NODES, HID), jnp.float32),
    )(x, w1, dinvb)


def _b_body(acc_ref, y1_ref, dinv_ref, b1_ref, w2_ref, y2_ref):
    dinv = dinv_ref[...]
    h = jnp.maximum(dinv * (acc_ref[0] + acc_ref[1] + y1_ref[...]) + b1_ref[...],
                    0.0)
    y2_ref[...] = jnp.dot(h, w2_ref[...],
                          preferred_element_type=jnp.float32) * dinv


def _dense_b(accp, y1, dinv, b1, w2):
    return pl.pallas_call(
        _b_body,
        grid=(GRID,),
        in_specs=[
            pl.BlockSpec((2, BLK, HID), lambda i: (0, i, 0)),
            pl.BlockSpec((BLK, HID), lambda i: (i, 0)),
            pl.BlockSpec((BLK, HID), lambda i: (i, 0)),
            pl.BlockSpec((1, HID), lambda i: (0, 0)),
            pl.BlockSpec((HID, HID), lambda i: (0, 0)),
        ],
        out_specs=pl.BlockSpec((BLK, HID), lambda i: (i, 0)),
        out_shape=jax.ShapeDtypeStruct((N_NODES, HID), jnp.float32),
    )(accp, y1, dinv, b1, w2)


def _c_body(acc_ref, y2_ref, dinv_ref, b2_ref, bat_ref, wh1_ref, bh1_ref,
            wh2_ref, bh2_ref, out_ref, sums, counts):
    i = pl.program_id(0)

    @pl.when(i == 0)
    def _():
        sums[...] = jnp.zeros_like(sums)
        counts[...] = jnp.zeros_like(counts)

    h = jnp.maximum(
        dinv_ref[...] * (acc_ref[0] + acc_ref[1] + y2_ref[...]) + b2_ref[...],
        0.0)
    onehot = (lax.broadcasted_iota(jnp.int32, (NUM_GRAPHS, BLK), 0)
              == bat_ref[0]).astype(jnp.float32)
    sums[...] += jnp.dot(onehot, h, preferred_element_type=jnp.float32)
    counts[...] += jnp.sum(onehot, axis=1, keepdims=True)

    @pl.when(i == pl.num_programs(0) - 1)
    def _():
        hg = sums[...] / jnp.maximum(counts[...], 1.0)
        z = jnp.maximum(
            jnp.dot(hg, wh1_ref[...], preferred_element_type=jnp.float32)
            + bh1_ref[...], 0.0)
        out_ref[...] = (jnp.dot(z, wh2_ref[...],
                                preferred_element_type=jnp.float32)
                        + bh2_ref[...])


def _dense_c(accp, y2, dinv, b2, bat, wh1, bh1, wh2, bh2):
    return pl.pallas_call(
        _c_body,
        grid=(GRID,),
        in_specs=[
            pl.BlockSpec((2, BLK, HID), lambda i: (0, i, 0)),
            pl.BlockSpec((BLK, HID), lambda i: (i, 0)),
            pl.BlockSpec((BLK, HID), lambda i: (i, 0)),
            pl.BlockSpec((1, HID), lambda i: (0, 0)),
            pl.BlockSpec((1, 1, BLK), lambda i: (i, 0, 0)),
            pl.BlockSpec((HID, HID // 2), lambda i: (0, 0)),
            pl.BlockSpec((1, HID // 2), lambda i: (0, 0)),
            pl.BlockSpec((HID // 2, 1), lambda i: (0, 0)),
            pl.BlockSpec((1, 1), lambda i: (0, 0)),
        ],
        out_specs=pl.BlockSpec((NUM_GRAPHS, 1), lambda i: (0, 0)),
        out_shape=jax.ShapeDtypeStruct((NUM_GRAPHS, 1), jnp.float32),
        scratch_shapes=[
            pltpu.VMEM((NUM_GRAPHS, HID), jnp.float32),
            pltpu.VMEM((NUM_GRAPHS, 1), jnp.float32),
        ],
    )(accp, y2, dinv, b2, bat, wh1, bh1, wh2, bh2)


def kernel(x, edge_index, batch, W1, b1, W2, b2, Wh1, bh1, Wh2, bh2):
    src = edge_index[0].astype(jnp.int32)
    dst = edge_index[1].astype(jnp.int32)
    srcp = jnp.concatenate([src, _PAD_SRC]).reshape(NW, NCH, CH)
    dstp = jnp.concatenate([dst, _PAD_DST]).reshape(NW, NCH, CH)
    z1 = jnp.zeros((ZROWS,), jnp.float32)
    z2 = jnp.zeros((ZROWS, HID), jnp.float32)

    degp = _deg_call(dstp, z1)
    dinv1d = lax.rsqrt(degp[:N_NODES] + degp[N_PAD:N_PAD + N_NODES] + 1.0)
    dinvb = jnp.broadcast_to(dinv1d[:, None], (N_NODES, HID))
    y1 = _dense_a(x, W1, dinvb)
    accp1 = _msg_call(y1, srcp, dstp, z2)
    y2 = _dense_b(accp1, y1, dinvb, b1.reshape(1, HID), W2)
    accp2 = _msg_call(y2, srcp, dstp, z2)
    out = _dense_c(accp2, y2, dinvb, b2.reshape(1, HID),
                   batch.astype(jnp.int32).reshape(GRID, 1, BLK),
                   Wh1, bh1.reshape(1, HID // 2), Wh2, bh2.reshape(1, 1))
    return out[:, 0]
